# Initial kernel scaffold; baseline (speedup 1.0000x reference)
import functools

import jax
import jax.numpy as jnp
from jax import lax
from jax.experimental import pallas as pl
from jax.experimental.pallas import tpu as pltpu
from jax.experimental.pallas import tpu_sc as plsc

N = 10000
E = 320000
NUM_CLASSES = 10
F = 16
NC, NS = 2, 16
NW = NC * NS
CHUNK = 128
CPT = 79
E_PAD = NW * CPT * CHUNK
N_PAD = 10240
RPT = N_PAD // NS

_MESH = dict(core_axis_name="c", subcore_axis_name="s")


@functools.partial(
    pl.kernel,
    out_type=jax.ShapeDtypeStruct((NC, N_PAD), jnp.float32),
    mesh=plsc.VectorSubcoreMesh(**_MESH),
    scratch_types=[
        pltpu.VMEM((CPT, CHUNK), jnp.int32),
        pltpu.VMEM((CHUNK,), jnp.float32),
        pltpu.VMEM((CHUNK,), jnp.float32),
        pltpu.VMEM_SHARED((N_PAD,), jnp.float32),
    ],
)
def _deg_pass(dst_hbm, out_hbm, dst_v, ones_v, zb_v, cnt):
    c = lax.axis_index("c")
    s = lax.axis_index("s")
    wid = c * NS + s
    pltpu.sync_copy(dst_hbm.at[wid], dst_v)
    for i in range(CHUNK // F):
        ones_v[pl.ds(i * F, F)] = jnp.ones((F,), jnp.float32)
        zb_v[pl.ds(i * F, F)] = jnp.zeros((F,), jnp.float32)
    for k in range(RPT // CHUNK):
        pltpu.sync_copy(zb_v, cnt.at[pl.ds(s * RPT + k * CHUNK, CHUNK)])
    plsc.subcore_barrier()

    def body(j, carry):
        pltpu.sync_copy(ones_v, cnt.at[dst_v.at[j]], add=True)
        return carry

    lax.fori_loop(0, CPT, body, 0)
    plsc.subcore_barrier()
    pltpu.sync_copy(cnt.at[pl.ds(s * RPT, RPT)],
                    out_hbm.at[c, pl.ds(s * RPT, RPT)])


@functools.partial(
    pl.kernel,
    out_type=jax.ShapeDtypeStruct((NC, N_PAD, F), jnp.float32),
    mesh=plsc.VectorSubcoreMesh(**_MESH),
    scratch_types=[
        pltpu.VMEM((CPT, CHUNK), jnp.int32),
        pltpu.VMEM((CPT, CHUNK), jnp.int32),
        pltpu.VMEM((CHUNK, F), jnp.float32),
        pltpu.VMEM((CHUNK, F), jnp.float32),
        pltpu.VMEM_SHARED((N_PAD, F), jnp.float32),
        pltpu.SemaphoreType.DMA,
    ],
)
def _edge_pass(h_hbm, src_hbm, dst_hbm, out_hbm,
               src_v, dst_v, rows_v, zb_v, acc, sem):
    c = lax.axis_index("c")
    s = lax.axis_index("s")
    wid = c * NS + s
    pltpu.sync_copy(src_hbm.at[wid], src_v)
    pltpu.sync_copy(dst_hbm.at[wid], dst_v)
    for i in range(CHUNK):
        zb_v[i, :] = jnp.zeros((F,), jnp.float32)
    for k in range(RPT // CHUNK):
        pltpu.sync_copy(zb_v, acc.at[pl.ds(s * RPT + k * CHUNK, CHUNK)])
    plsc.subcore_barrier()

    def body(j, carry):
        pltpu.async_copy(h_hbm.at[src_v.at[j]], rows_v, sem).wait()
        pltpu.sync_copy(rows_v, acc.at[dst_v.at[j]], add=True)
        return carry

    lax.fori_loop(0, CPT, body, 0)
    plsc.subcore_barrier()
    pltpu.sync_copy(acc.at[pl.ds(s * RPT, RPT)],
                    out_hbm.at[c, pl.ds(s * RPT, RPT)])


def _tc1(cnt_ref, x_ref, w1_ref, hs_ref):
    cnt = cnt_ref[0, :] + cnt_ref[1, :]
    dinv = lax.rsqrt(cnt + 1.0)[:, None]
    h = jnp.dot(x_ref[:, :], w1_ref[:, :], preferred_element_type=jnp.float32)
    hs_ref[:, :] = h * dinv


def _tc2(cnt_ref, acc_ref, hs1_ref, b1_ref, w2_ref, hs2_ref):
    cnt = cnt_ref[0, :] + cnt_ref[1, :]
    dinv = lax.rsqrt(cnt + 1.0)[:, None]
    out1 = dinv * (acc_ref[0] + acc_ref[1] + hs1_ref[:, :]) + b1_ref[:][None, :]
    h2 = jnp.maximum(out1, 0.0)
    hs2_ref[:, :] = jnp.dot(h2, w2_ref[:, :],
                            preferred_element_type=jnp.float32) * dinv


def _tc3(cnt_ref, acc_ref, hs2_ref, b2_ref, w3_ref, embed_ref, hs3_ref):
    cnt = cnt_ref[0, :] + cnt_ref[1, :]
    dinv = lax.rsqrt(cnt + 1.0)[:, None]
    out2 = dinv * (acc_ref[0] + acc_ref[1] + hs2_ref[:, :]) + b2_ref[:][None, :]
    nrm = jnp.sqrt(jnp.sum(out2 * out2, axis=1, keepdims=True))
    embed = out2 / jnp.maximum(nrm, 1e-12)
    embed_ref[:, :] = embed
    hs3_ref[:, :] = jnp.dot(embed, w3_ref[:, :],
                            preferred_element_type=jnp.float32) * dinv


def _tc4(cnt_ref, acc_ref, hs3_ref, b3_ref, wp_ref, bp_ref, wl_ref, bl_ref,
         h_ref, logp_ref):
    cnt = cnt_ref[0, :] + cnt_ref[1, :]
    dinv = lax.rsqrt(cnt + 1.0)[:, None]
    out3 = dinv * (acc_ref[0] + acc_ref[1] + hs3_ref[:, :]) + b3_ref[:][None, :]
    hfull = jnp.dot(out3, wp_ref[:, :],
                    preferred_element_type=jnp.float32) + bp_ref[:][None, :]
    rid = lax.broadcasted_iota(jnp.int32, (N_PAD, F), 0)
    hmask = jnp.where(rid < N, hfull, 0.0)
    h_ref[:, :] = hmask
    pooled = jnp.sum(hmask, axis=0, keepdims=True) * (1.0 / N)
    logits = jnp.dot(pooled, wl_ref[:, :],
                     preferred_element_type=jnp.float32) + bl_ref[:][None, :]
    cid = lax.broadcasted_iota(jnp.int32, (1, F), 1)
    valid = cid < NUM_CLASSES
    m = jnp.max(jnp.where(valid, logits, -1e30), axis=1, keepdims=True)
    ex = jnp.where(valid, jnp.exp(logits - m), 0.0)
    lse = jnp.log(jnp.sum(ex, axis=1, keepdims=True))
    logp_ref[:, :] = logits - m - lse


_SD = jax.ShapeDtypeStruct
_tc1_call = pl.pallas_call(_tc1, out_shape=_SD((N_PAD, F), jnp.float32))
_tc2_call = pl.pallas_call(_tc2, out_shape=_SD((N_PAD, F), jnp.float32))
_tc3_call = pl.pallas_call(
    _tc3, out_shape=[_SD((N_PAD, F), jnp.float32), _SD((N_PAD, F), jnp.float32)])
_tc4_call = pl.pallas_call(
    _tc4, out_shape=[_SD((N_PAD, F), jnp.float32), _SD((1, F), jnp.float32)])


def kernel(x, edge_index, W1, b1, W2, b2, W3, b3, Wp, bp, Wl, bl):
    ei = edge_index.astype(jnp.int32)
    pad = N + (jnp.arange(E_PAD - E, dtype=jnp.int32) % (N_PAD - N))
    srcp = jnp.concatenate([ei[0], pad]).reshape(NW, CPT, CHUNK)
    dstp = jnp.concatenate([ei[1], pad]).reshape(NW, CPT, CHUNK)
    x_pad = jnp.pad(x, ((0, N_PAD - N), (0, 0)))
    w2p = jnp.pad(W2, ((0, 0), (0, F - W2.shape[1])))
    b2p = jnp.pad(b2, (0, F - b2.shape[0]))
    w3p = jnp.pad(W3, ((0, F - W3.shape[0]), (0, F - W3.shape[1])))
    b3p = jnp.pad(b3, (0, F - b3.shape[0]))
    wpp = jnp.pad(Wp, ((0, F - Wp.shape[0]), (0, F - Wp.shape[1])))
    bpp = jnp.pad(bp, (0, F - bp.shape[0]))
    wlp = jnp.pad(Wl, ((0, F - Wl.shape[0]), (0, F - Wl.shape[1])))
    blp = jnp.pad(bl, (0, F - bl.shape[0]))

    counts = _deg_pass(dstp)
    hs1 = _tc1_call(counts, x_pad, W1)
    acc1 = _edge_pass(hs1, srcp, dstp)
    hs2 = _tc2_call(counts, acc1, hs1, b1, w2p)
    acc2 = _edge_pass(hs2, srcp, dstp)
    embed, hs3 = _tc3_call(counts, acc2, hs2, b2p, w3p)
    acc3 = _edge_pass(hs3, srcp, dstp)
    h_full, logp = _tc4_call(counts, acc3, hs3, b3p, wpp, bpp, wlp, blp)
    return (logp[:, :NUM_CLASSES], (h_full[:N, :3], embed[:N, :3]))


# speedup vs baseline: 32.8548x; 32.8548x over previous
import functools

import jax
import jax.numpy as jnp
from jax import lax
from jax.experimental import pallas as pl
from jax.experimental.pallas import tpu as pltpu
from jax.experimental.pallas import tpu_sc as plsc

N = 10000
E = 320000
NUM_CLASSES = 10
F = 16
NC, NS = 2, 16
NW = NC * NS
CHUNK = 128
CPT = 79
E_PAD = NW * CPT * CHUNK
N_PAD = 10240
RPT = N_PAD // NS

_MESH = dict(core_axis_name="c", subcore_axis_name="s")


@functools.partial(
    pl.kernel,
    out_type=jax.ShapeDtypeStruct((NC, N_PAD), jnp.float32),
    mesh=plsc.VectorSubcoreMesh(**_MESH),
    scratch_types=[
        pltpu.VMEM((CPT, CHUNK), jnp.int32),
        pltpu.VMEM((CHUNK,), jnp.float32),
        pltpu.VMEM((CHUNK,), jnp.float32),
        pltpu.VMEM_SHARED((N_PAD,), jnp.float32),
    ],
    compiler_params=pltpu.CompilerParams(use_tc_tiling_on_sc=False),
)
def _deg_pass(dst_hbm, out_hbm, dst_v, ones_v, zb_v, cnt):
    c = lax.axis_index("c")
    s = lax.axis_index("s")
    wid = c * NS + s
    pltpu.sync_copy(dst_hbm.at[wid], dst_v)
    for i in range(CHUNK // F):
        ones_v[pl.ds(i * F, F)] = jnp.ones((F,), jnp.float32)
        zb_v[pl.ds(i * F, F)] = jnp.zeros((F,), jnp.float32)
    for k in range(RPT // CHUNK):
        pltpu.sync_copy(zb_v, cnt.at[pl.ds(s * RPT + k * CHUNK, CHUNK)])
    plsc.subcore_barrier()

    def body(j, carry):
        pltpu.sync_copy(ones_v, cnt.at[dst_v.at[j]], add=True)
        return carry

    lax.fori_loop(0, CPT, body, 0)
    plsc.subcore_barrier()
    pltpu.sync_copy(cnt.at[pl.ds(s * RPT, RPT)],
                    out_hbm.at[c, pl.ds(s * RPT, RPT)])


@functools.partial(
    pl.kernel,
    out_type=jax.ShapeDtypeStruct((NC, N_PAD, F), jnp.float32),
    mesh=plsc.VectorSubcoreMesh(**_MESH),
    scratch_types=[
        pltpu.VMEM((CPT, CHUNK), jnp.int32),
        pltpu.VMEM((CPT, CHUNK), jnp.int32),
        pltpu.VMEM((CHUNK, F), jnp.float32),
        pltpu.VMEM((CHUNK, F), jnp.float32),
        pltpu.VMEM_SHARED((N_PAD, F), jnp.float32),
        pltpu.SemaphoreType.DMA,
    ],
    compiler_params=pltpu.CompilerParams(use_tc_tiling_on_sc=False),
)
def _edge_pass(h_hbm, src_hbm, dst_hbm, out_hbm,
               src_v, dst_v, rows_v, zb_v, acc, sem):
    c = lax.axis_index("c")
    s = lax.axis_index("s")
    wid = c * NS + s
    pltpu.sync_copy(src_hbm.at[wid], src_v)
    pltpu.sync_copy(dst_hbm.at[wid], dst_v)
    for i in range(CHUNK):
        zb_v[i, :] = jnp.zeros((F,), jnp.float32)
    for k in range(RPT // CHUNK):
        pltpu.sync_copy(zb_v, acc.at[pl.ds(s * RPT + k * CHUNK, CHUNK)])
    plsc.subcore_barrier()

    def body(j, carry):
        pltpu.async_copy(h_hbm.at[src_v.at[j]], rows_v, sem).wait()
        pltpu.sync_copy(rows_v, acc.at[dst_v.at[j]], add=True)
        return carry

    lax.fori_loop(0, CPT, body, 0)
    plsc.subcore_barrier()
    pltpu.sync_copy(acc.at[pl.ds(s * RPT, RPT)],
                    out_hbm.at[c, pl.ds(s * RPT, RPT)])


def _tc1(cnt_ref, x_ref, w1_ref, hs_ref):
    cnt = cnt_ref[0, :] + cnt_ref[1, :]
    dinv = lax.rsqrt(cnt + 1.0)[:, None]
    h = jnp.dot(x_ref[:, :], w1_ref[:, :], preferred_element_type=jnp.float32)
    hs_ref[:, :] = h * dinv


def _tc2(cnt_ref, acc_ref, hs1_ref, b1_ref, w2_ref, hs2_ref):
    cnt = cnt_ref[0, :] + cnt_ref[1, :]
    dinv = lax.rsqrt(cnt + 1.0)[:, None]
    out1 = dinv * (acc_ref[0] + acc_ref[1] + hs1_ref[:, :]) + b1_ref[:][None, :]
    h2 = jnp.maximum(out1, 0.0)
    hs2_ref[:, :] = jnp.dot(h2, w2_ref[:, :],
                            preferred_element_type=jnp.float32) * dinv


def _tc3(cnt_ref, acc_ref, hs2_ref, b2_ref, w3_ref, embed_ref, hs3_ref):
    cnt = cnt_ref[0, :] + cnt_ref[1, :]
    dinv = lax.rsqrt(cnt + 1.0)[:, None]
    out2 = dinv * (acc_ref[0] + acc_ref[1] + hs2_ref[:, :]) + b2_ref[:][None, :]
    nrm = jnp.sqrt(jnp.sum(out2 * out2, axis=1, keepdims=True))
    embed = out2 / jnp.maximum(nrm, 1e-12)
    embed_ref[:, :] = embed
    hs3_ref[:, :] = jnp.dot(embed, w3_ref[:, :],
                            preferred_element_type=jnp.float32) * dinv


def _tc4(cnt_ref, acc_ref, hs3_ref, b3_ref, wp_ref, bp_ref, wl_ref, bl_ref,
         h_ref, logp_ref):
    cnt = cnt_ref[0, :] + cnt_ref[1, :]
    dinv = lax.rsqrt(cnt + 1.0)[:, None]
    out3 = dinv * (acc_ref[0] + acc_ref[1] + hs3_ref[:, :]) + b3_ref[:][None, :]
    hfull = jnp.dot(out3, wp_ref[:, :],
                    preferred_element_type=jnp.float32) + bp_ref[:][None, :]
    rid = lax.broadcasted_iota(jnp.int32, (N_PAD, F), 0)
    hmask = jnp.where(rid < N, hfull, 0.0)
    h_ref[:, :] = hmask
    pooled = jnp.sum(hmask, axis=0, keepdims=True) * (1.0 / N)
    logits = jnp.dot(pooled, wl_ref[:, :],
                     preferred_element_type=jnp.float32) + bl_ref[:][None, :]
    cid = lax.broadcasted_iota(jnp.int32, (1, F), 1)
    valid = cid < NUM_CLASSES
    m = jnp.max(jnp.where(valid, logits, -1e30), axis=1, keepdims=True)
    ex = jnp.where(valid, jnp.exp(logits - m), 0.0)
    lse = jnp.log(jnp.sum(ex, axis=1, keepdims=True))
    logp_ref[:, :] = logits - m - lse


_SD = jax.ShapeDtypeStruct
_tc1_call = pl.pallas_call(_tc1, out_shape=_SD((N_PAD, F), jnp.float32))
_tc2_call = pl.pallas_call(_tc2, out_shape=_SD((N_PAD, F), jnp.float32))
_tc3_call = pl.pallas_call(
    _tc3, out_shape=[_SD((N_PAD, F), jnp.float32), _SD((N_PAD, F), jnp.float32)])
_tc4_call = pl.pallas_call(
    _tc4, out_shape=[_SD((N_PAD, F), jnp.float32), _SD((1, F), jnp.float32)])


def kernel(x, edge_index, W1, b1, W2, b2, W3, b3, Wp, bp, Wl, bl):
    ei = edge_index.astype(jnp.int32)
    pad = N + (jnp.arange(E_PAD - E, dtype=jnp.int32) % (N_PAD - N))
    srcp = jnp.concatenate([ei[0], pad]).reshape(NW, CPT, CHUNK)
    dstp = jnp.concatenate([ei[1], pad]).reshape(NW, CPT, CHUNK)
    x_pad = jnp.pad(x, ((0, N_PAD - N), (0, 0)))
    w2p = jnp.pad(W2, ((0, 0), (0, F - W2.shape[1])))
    b2p = jnp.pad(b2, (0, F - b2.shape[0]))
    w3p = jnp.pad(W3, ((0, F - W3.shape[0]), (0, F - W3.shape[1])))
    b3p = jnp.pad(b3, (0, F - b3.shape[0]))
    wpp = jnp.pad(Wp, ((0, F - Wp.shape[0]), (0, F - Wp.shape[1])))
    bpp = jnp.pad(bp, (0, F - bp.shape[0]))
    wlp = jnp.pad(Wl, ((0, F - Wl.shape[0]), (0, F - Wl.shape[1])))
    blp = jnp.pad(bl, (0, F - bl.shape[0]))

    counts = _deg_pass(dstp)
    hs1 = _tc1_call(counts, x_pad, W1)
    acc1 = _edge_pass(hs1, srcp, dstp)
    hs2 = _tc2_call(counts, acc1, hs1, b1, w2p)
    acc2 = _edge_pass(hs2, srcp, dstp)
    embed, hs3 = _tc3_call(counts, acc2, hs2, b2p, w3p)
    acc3 = _edge_pass(hs3, srcp, dstp)
    h_full, logp = _tc4_call(counts, acc3, hs3, b3p, wpp, bpp, wlp, blp)
    return (logp[:, :NUM_CLASSES], (h_full[:N, :3], embed[:N, :3]))


# speedup vs baseline: 62.0969x; 1.8900x over previous
import functools

import jax
import jax.numpy as jnp
from jax import lax
from jax.experimental import pallas as pl
from jax.experimental.pallas import tpu as pltpu
from jax.experimental.pallas import tpu_sc as plsc

N = 10000
E = 320000
NUM_CLASSES = 10
F = 16
NC, NS = 2, 16
NW = NC * NS
CHUNK = 128
CPT = 79
E_PAD = NW * CPT * CHUNK
N_PAD = 10240
RPT = N_PAD // NS
LEAD = 8
NRING = 2 * LEAD

_MESH = dict(core_axis_name="c", subcore_axis_name="s")


@functools.partial(
    pl.kernel,
    out_type=jax.ShapeDtypeStruct((NC, N_PAD), jnp.float32),
    mesh=plsc.VectorSubcoreMesh(**_MESH),
    scratch_types=[
        pltpu.VMEM((CPT, CHUNK), jnp.int32),
        pltpu.VMEM((CHUNK,), jnp.float32),
        pltpu.VMEM((CHUNK,), jnp.float32),
        pltpu.VMEM_SHARED((N_PAD,), jnp.float32),
        pltpu.SemaphoreType.DMA,
    ],
    compiler_params=pltpu.CompilerParams(use_tc_tiling_on_sc=False),
)
def _deg_pass(dst_hbm, out_hbm, dst_v, ones_v, zb_v, cnt, sem):
    c = lax.axis_index("c")
    s = lax.axis_index("s")
    wid = c * NS + s
    pltpu.sync_copy(dst_hbm.at[wid], dst_v)
    for i in range(CHUNK // F):
        ones_v[pl.ds(i * F, F)] = jnp.ones((F,), jnp.float32)
        zb_v[pl.ds(i * F, F)] = jnp.zeros((F,), jnp.float32)
    for k in range(RPT // CHUNK):
        pltpu.sync_copy(zb_v, cnt.at[pl.ds(s * RPT + k * CHUNK, CHUNK)])
    plsc.subcore_barrier()

    def body(j, carry):
        pltpu.async_copy(ones_v, cnt.at[dst_v.at[j]], sem, add=True)
        return carry

    lax.fori_loop(0, CPT, body, 0)

    def drain(j, carry):
        pltpu.make_async_copy(out_hbm.at[0, pl.ds(0, CHUNK)], zb_v, sem).wait()
        return carry

    lax.fori_loop(0, CPT, drain, 0)
    plsc.subcore_barrier()
    pltpu.sync_copy(cnt.at[pl.ds(s * RPT, RPT)],
                    out_hbm.at[c, pl.ds(s * RPT, RPT)])


@functools.partial(
    pl.kernel,
    out_type=jax.ShapeDtypeStruct((NC, N_PAD, F), jnp.float32),
    mesh=plsc.VectorSubcoreMesh(**_MESH),
    scratch_types=[
        pltpu.VMEM((CPT, CHUNK), jnp.int32),
        pltpu.VMEM((CPT, CHUNK), jnp.int32),
        pltpu.VMEM((NRING, CHUNK, F), jnp.float32),
        pltpu.VMEM((CHUNK, F), jnp.float32),
        pltpu.VMEM_SHARED((N_PAD, F), jnp.float32),
        pltpu.SemaphoreType.DMA,
        pltpu.SemaphoreType.DMA,
    ],
    compiler_params=pltpu.CompilerParams(use_tc_tiling_on_sc=False),
)
def _edge_pass(h_hbm, src_hbm, dst_hbm, out_hbm,
               src_v, dst_v, rows_v, zb_v, acc, sem_g, sem_s):
    c = lax.axis_index("c")
    s = lax.axis_index("s")
    wid = c * NS + s
    pltpu.sync_copy(src_hbm.at[wid], src_v)
    pltpu.sync_copy(dst_hbm.at[wid], dst_v)
    for i in range(CHUNK):
        zb_v[i, :] = jnp.zeros((F,), jnp.float32)
    for k in range(RPT // CHUNK):
        pltpu.sync_copy(zb_v, acc.at[pl.ds(s * RPT + k * CHUNK, CHUNK)])
    plsc.subcore_barrier()

    for b in range(LEAD):
        pltpu.async_copy(h_hbm.at[src_v.at[b]], rows_v.at[b], sem_g)

    def body(j, carry):
        pltpu.make_async_copy(out_hbm.at[0, pl.ds(0, CHUNK)],
                              rows_v.at[0], sem_g).wait()
        pltpu.async_copy(rows_v.at[j % NRING], acc.at[dst_v.at[j]],
                         sem_s, add=True)

        @pl.when(j >= LEAD)
        def _():
            pltpu.make_async_copy(out_hbm.at[0, pl.ds(0, CHUNK)],
                                  rows_v.at[0], sem_s).wait()

        @pl.when(j + LEAD < CPT)
        def _():
            jn = jnp.minimum(j + LEAD, CPT - 1)
            pltpu.async_copy(h_hbm.at[src_v.at[jn]],
                             rows_v.at[jn % NRING], sem_g)
        return carry

    lax.fori_loop(0, CPT, body, 0)
    for b in range(LEAD):
        pltpu.make_async_copy(out_hbm.at[0, pl.ds(0, CHUNK)],
                              rows_v.at[0], sem_s).wait()
    plsc.subcore_barrier()
    pltpu.sync_copy(acc.at[pl.ds(s * RPT, RPT)],
                    out_hbm.at[c, pl.ds(s * RPT, RPT)])


def _tc1(cnt_ref, x_ref, w1_ref, hs_ref):
    cnt = cnt_ref[0, :] + cnt_ref[1, :]
    dinv = lax.rsqrt(cnt + 1.0)[:, None]
    h = jnp.dot(x_ref[:, :], w1_ref[:, :], preferred_element_type=jnp.float32)
    hs_ref[:, :] = h * dinv


def _tc2(cnt_ref, acc_ref, hs1_ref, b1_ref, w2_ref, hs2_ref):
    cnt = cnt_ref[0, :] + cnt_ref[1, :]
    dinv = lax.rsqrt(cnt + 1.0)[:, None]
    out1 = dinv * (acc_ref[0] + acc_ref[1] + hs1_ref[:, :]) + b1_ref[:][None, :]
    h2 = jnp.maximum(out1, 0.0)
    hs2_ref[:, :] = jnp.dot(h2, w2_ref[:, :],
                            preferred_element_type=jnp.float32) * dinv


def _tc3(cnt_ref, acc_ref, hs2_ref, b2_ref, w3_ref, embed_ref, hs3_ref):
    cnt = cnt_ref[0, :] + cnt_ref[1, :]
    dinv = lax.rsqrt(cnt + 1.0)[:, None]
    out2 = dinv * (acc_ref[0] + acc_ref[1] + hs2_ref[:, :]) + b2_ref[:][None, :]
    nrm = jnp.sqrt(jnp.sum(out2 * out2, axis=1, keepdims=True))
    embed = out2 / jnp.maximum(nrm, 1e-12)
    embed_ref[:, :] = embed
    hs3_ref[:, :] = jnp.dot(embed, w3_ref[:, :],
                            preferred_element_type=jnp.float32) * dinv


def _tc4(cnt_ref, acc_ref, hs3_ref, b3_ref, wp_ref, bp_ref, wl_ref, bl_ref,
         h_ref, logp_ref):
    cnt = cnt_ref[0, :] + cnt_ref[1, :]
    dinv = lax.rsqrt(cnt + 1.0)[:, None]
    out3 = dinv * (acc_ref[0] + acc_ref[1] + hs3_ref[:, :]) + b3_ref[:][None, :]
    hfull = jnp.dot(out3, wp_ref[:, :],
                    preferred_element_type=jnp.float32) + bp_ref[:][None, :]
    rid = lax.broadcasted_iota(jnp.int32, (N_PAD, F), 0)
    hmask = jnp.where(rid < N, hfull, 0.0)
    h_ref[:, :] = hmask
    pooled = jnp.sum(hmask, axis=0, keepdims=True) * (1.0 / N)
    logits = jnp.dot(pooled, wl_ref[:, :],
                     preferred_element_type=jnp.float32) + bl_ref[:][None, :]
    cid = lax.broadcasted_iota(jnp.int32, (1, F), 1)
    valid = cid < NUM_CLASSES
    m = jnp.max(jnp.where(valid, logits, -1e30), axis=1, keepdims=True)
    ex = jnp.where(valid, jnp.exp(logits - m), 0.0)
    lse = jnp.log(jnp.sum(ex, axis=1, keepdims=True))
    logp_ref[:, :] = logits - m - lse


_SD = jax.ShapeDtypeStruct
_tc1_call = pl.pallas_call(_tc1, out_shape=_SD((N_PAD, F), jnp.float32))
_tc2_call = pl.pallas_call(_tc2, out_shape=_SD((N_PAD, F), jnp.float32))
_tc3_call = pl.pallas_call(
    _tc3, out_shape=[_SD((N_PAD, F), jnp.float32), _SD((N_PAD, F), jnp.float32)])
_tc4_call = pl.pallas_call(
    _tc4, out_shape=[_SD((N_PAD, F), jnp.float32), _SD((1, F), jnp.float32)])


def kernel(x, edge_index, W1, b1, W2, b2, W3, b3, Wp, bp, Wl, bl):
    ei = edge_index.astype(jnp.int32)
    pad = N + (jnp.arange(E_PAD - E, dtype=jnp.int32) % (N_PAD - N))
    srcp = jnp.concatenate([ei[0], pad]).reshape(NW, CPT, CHUNK)
    dstp = jnp.concatenate([ei[1], pad]).reshape(NW, CPT, CHUNK)
    x_pad = jnp.pad(x, ((0, N_PAD - N), (0, 0)))
    w2p = jnp.pad(W2, ((0, 0), (0, F - W2.shape[1])))
    b2p = jnp.pad(b2, (0, F - b2.shape[0]))
    w3p = jnp.pad(W3, ((0, F - W3.shape[0]), (0, F - W3.shape[1])))
    b3p = jnp.pad(b3, (0, F - b3.shape[0]))
    wpp = jnp.pad(Wp, ((0, F - Wp.shape[0]), (0, F - Wp.shape[1])))
    bpp = jnp.pad(bp, (0, F - bp.shape[0]))
    wlp = jnp.pad(Wl, ((0, F - Wl.shape[0]), (0, F - Wl.shape[1])))
    blp = jnp.pad(bl, (0, F - bl.shape[0]))

    counts = _deg_pass(dstp)
    hs1 = _tc1_call(counts, x_pad, W1)
    acc1 = _edge_pass(hs1, srcp, dstp)
    hs2 = _tc2_call(counts, acc1, hs1, b1, w2p)
    acc2 = _edge_pass(hs2, srcp, dstp)
    embed, hs3 = _tc3_call(counts, acc2, hs2, b2p, w3p)
    acc3 = _edge_pass(hs3, srcp, dstp)
    h_full, logp = _tc4_call(counts, acc3, hs3, b3p, wpp, bpp, wlp, blp)
    return (logp[:, :NUM_CLASSES], (h_full[:N, :3], embed[:N, :3]))


# speedup vs baseline: 64.7210x; 1.0423x over previous
import functools

import jax
import jax.numpy as jnp
from jax import lax
from jax.experimental import pallas as pl
from jax.experimental.pallas import tpu as pltpu
from jax.experimental.pallas import tpu_sc as plsc

N = 10000
E = 320000
NUM_CLASSES = 10
F = 16
NC, NS = 2, 16
NW = NC * NS
CHUNK = 128
CPT = 79
E_PAD = NW * CPT * CHUNK
N_PAD = 10240
RPT = N_PAD // NS
LEAD = 8
NRING = 2 * LEAD

_MESH = dict(core_axis_name="c", subcore_axis_name="s")


@functools.partial(
    pl.kernel,
    out_type=jax.ShapeDtypeStruct((NC, N_PAD), jnp.float32),
    mesh=plsc.VectorSubcoreMesh(**_MESH),
    scratch_types=[
        pltpu.VMEM((CPT, CHUNK), jnp.int32),
        pltpu.VMEM((CHUNK,), jnp.float32),
        pltpu.VMEM((CHUNK,), jnp.float32),
        pltpu.VMEM_SHARED((N_PAD,), jnp.float32),
        pltpu.SemaphoreType.DMA,
    ],
    compiler_params=pltpu.CompilerParams(use_tc_tiling_on_sc=False),
)
def _deg_pass(dst_hbm, out_hbm, dst_v, ones_v, zb_v, cnt, sem):
    c = lax.axis_index("c")
    s = lax.axis_index("s")
    wid = c * NS + s
    pltpu.sync_copy(dst_hbm.at[wid], dst_v)
    for i in range(CHUNK // F):
        ones_v[pl.ds(i * F, F)] = jnp.ones((F,), jnp.float32)
        zb_v[pl.ds(i * F, F)] = jnp.zeros((F,), jnp.float32)
    for k in range(RPT // CHUNK):
        pltpu.sync_copy(zb_v, cnt.at[pl.ds(s * RPT + k * CHUNK, CHUNK)])
    plsc.subcore_barrier()

    def body(j, carry):
        pltpu.async_copy(ones_v, cnt.at[dst_v.at[j]], sem, add=True)
        return carry

    lax.fori_loop(0, CPT, body, 0)

    def drain(j, carry):
        pltpu.make_async_copy(out_hbm.at[0, pl.ds(0, CHUNK)], zb_v, sem).wait()
        return carry

    lax.fori_loop(0, CPT, drain, 0)
    plsc.subcore_barrier()
    pltpu.sync_copy(cnt.at[pl.ds(s * RPT, RPT)],
                    out_hbm.at[c, pl.ds(s * RPT, RPT)])


@functools.partial(
    pl.kernel,
    out_type=jax.ShapeDtypeStruct((NC, N_PAD, F), jnp.float32),
    mesh=plsc.VectorSubcoreMesh(**_MESH),
    scratch_types=[
        pltpu.VMEM((CPT, CHUNK), jnp.int32),
        pltpu.VMEM((CPT, CHUNK), jnp.int32),
        pltpu.VMEM((NRING, CHUNK, F), jnp.float32),
        pltpu.VMEM((CHUNK, F), jnp.float32),
        pltpu.VMEM_SHARED((N_PAD, F), jnp.float32),
        pltpu.VMEM_SHARED((N_PAD, F), jnp.float32),
        pltpu.SemaphoreType.DMA,
        pltpu.SemaphoreType.DMA,
    ],
    compiler_params=pltpu.CompilerParams(use_tc_tiling_on_sc=False),
)
def _edge_pass(h_hbm, src_hbm, dst_hbm, out_hbm,
               src_v, dst_v, rows_v, zb_v, acc, h_stage, sem_g, sem_s):
    c = lax.axis_index("c")
    s = lax.axis_index("s")
    wid = c * NS + s
    pltpu.sync_copy(src_hbm.at[wid], src_v)
    pltpu.sync_copy(dst_hbm.at[wid], dst_v)
    pltpu.sync_copy(h_hbm.at[pl.ds(s * RPT, RPT)],
                    h_stage.at[pl.ds(s * RPT, RPT)])
    for i in range(CHUNK):
        zb_v[i, :] = jnp.zeros((F,), jnp.float32)
    for k in range(RPT // CHUNK):
        pltpu.sync_copy(zb_v, acc.at[pl.ds(s * RPT + k * CHUNK, CHUNK)])
    plsc.subcore_barrier()

    for b in range(LEAD):
        pltpu.async_copy(h_stage.at[src_v.at[b]], rows_v.at[b], sem_g)

    def body(j, carry):
        pltpu.make_async_copy(out_hbm.at[0, pl.ds(0, CHUNK)],
                              rows_v.at[0], sem_g).wait()
        pltpu.async_copy(rows_v.at[j % NRING], acc.at[dst_v.at[j]],
                         sem_s, add=True)

        @pl.when(j >= LEAD)
        def _():
            pltpu.make_async_copy(out_hbm.at[0, pl.ds(0, CHUNK)],
                                  rows_v.at[0], sem_s).wait()

        @pl.when(j + LEAD < CPT)
        def _():
            jn = jnp.minimum(j + LEAD, CPT - 1)
            pltpu.async_copy(h_stage.at[src_v.at[jn]],
                             rows_v.at[jn % NRING], sem_g)
        return carry

    lax.fori_loop(0, CPT, body, 0)
    for b in range(LEAD):
        pltpu.make_async_copy(out_hbm.at[0, pl.ds(0, CHUNK)],
                              rows_v.at[0], sem_s).wait()
    plsc.subcore_barrier()
    pltpu.sync_copy(acc.at[pl.ds(s * RPT, RPT)],
                    out_hbm.at[c, pl.ds(s * RPT, RPT)])


def _tc1(cnt_ref, x_ref, w1_ref, hs_ref):
    cnt = cnt_ref[0, :] + cnt_ref[1, :]
    dinv = lax.rsqrt(cnt + 1.0)[:, None]
    h = jnp.dot(x_ref[:, :], w1_ref[:, :], preferred_element_type=jnp.float32)
    hs_ref[:, :] = h * dinv


def _tc2(cnt_ref, acc_ref, hs1_ref, b1_ref, w2_ref, hs2_ref):
    cnt = cnt_ref[0, :] + cnt_ref[1, :]
    dinv = lax.rsqrt(cnt + 1.0)[:, None]
    out1 = dinv * (acc_ref[0] + acc_ref[1] + hs1_ref[:, :]) + b1_ref[:][None, :]
    h2 = jnp.maximum(out1, 0.0)
    hs2_ref[:, :] = jnp.dot(h2, w2_ref[:, :],
                            preferred_element_type=jnp.float32) * dinv


def _tc3(cnt_ref, acc_ref, hs2_ref, b2_ref, w3_ref, embed_ref, hs3_ref):
    cnt = cnt_ref[0, :] + cnt_ref[1, :]
    dinv = lax.rsqrt(cnt + 1.0)[:, None]
    out2 = dinv * (acc_ref[0] + acc_ref[1] + hs2_ref[:, :]) + b2_ref[:][None, :]
    nrm = jnp.sqrt(jnp.sum(out2 * out2, axis=1, keepdims=True))
    embed = out2 / jnp.maximum(nrm, 1e-12)
    embed_ref[:, :] = embed
    hs3_ref[:, :] = jnp.dot(embed, w3_ref[:, :],
                            preferred_element_type=jnp.float32) * dinv


def _tc4(cnt_ref, acc_ref, hs3_ref, b3_ref, wp_ref, bp_ref, wl_ref, bl_ref,
         h_ref, logp_ref):
    cnt = cnt_ref[0, :] + cnt_ref[1, :]
    dinv = lax.rsqrt(cnt + 1.0)[:, None]
    out3 = dinv * (acc_ref[0] + acc_ref[1] + hs3_ref[:, :]) + b3_ref[:][None, :]
    hfull = jnp.dot(out3, wp_ref[:, :],
                    preferred_element_type=jnp.float32) + bp_ref[:][None, :]
    rid = lax.broadcasted_iota(jnp.int32, (N_PAD, F), 0)
    hmask = jnp.where(rid < N, hfull, 0.0)
    h_ref[:, :] = hmask
    pooled = jnp.sum(hmask, axis=0, keepdims=True) * (1.0 / N)
    logits = jnp.dot(pooled, wl_ref[:, :],
                     preferred_element_type=jnp.float32) + bl_ref[:][None, :]
    cid = lax.broadcasted_iota(jnp.int32, (1, F), 1)
    valid = cid < NUM_CLASSES
    m = jnp.max(jnp.where(valid, logits, -1e30), axis=1, keepdims=True)
    ex = jnp.where(valid, jnp.exp(logits - m), 0.0)
    lse = jnp.log(jnp.sum(ex, axis=1, keepdims=True))
    logp_ref[:, :] = logits - m - lse


_SD = jax.ShapeDtypeStruct
_tc1_call = pl.pallas_call(_tc1, out_shape=_SD((N_PAD, F), jnp.float32))
_tc2_call = pl.pallas_call(_tc2, out_shape=_SD((N_PAD, F), jnp.float32))
_tc3_call = pl.pallas_call(
    _tc3, out_shape=[_SD((N_PAD, F), jnp.float32), _SD((N_PAD, F), jnp.float32)])
_tc4_call = pl.pallas_call(
    _tc4, out_shape=[_SD((N_PAD, F), jnp.float32), _SD((1, F), jnp.float32)])


def kernel(x, edge_index, W1, b1, W2, b2, W3, b3, Wp, bp, Wl, bl):
    ei = edge_index.astype(jnp.int32)
    pad = N + (jnp.arange(E_PAD - E, dtype=jnp.int32) % (N_PAD - N))
    srcp = jnp.concatenate([ei[0], pad]).reshape(NW, CPT, CHUNK)
    dstp = jnp.concatenate([ei[1], pad]).reshape(NW, CPT, CHUNK)
    x_pad = jnp.pad(x, ((0, N_PAD - N), (0, 0)))
    w2p = jnp.pad(W2, ((0, 0), (0, F - W2.shape[1])))
    b2p = jnp.pad(b2, (0, F - b2.shape[0]))
    w3p = jnp.pad(W3, ((0, F - W3.shape[0]), (0, F - W3.shape[1])))
    b3p = jnp.pad(b3, (0, F - b3.shape[0]))
    wpp = jnp.pad(Wp, ((0, F - Wp.shape[0]), (0, F - Wp.shape[1])))
    bpp = jnp.pad(bp, (0, F - bp.shape[0]))
    wlp = jnp.pad(Wl, ((0, F - Wl.shape[0]), (0, F - Wl.shape[1])))
    blp = jnp.pad(bl, (0, F - bl.shape[0]))

    counts = _deg_pass(dstp)
    hs1 = _tc1_call(counts, x_pad, W1)
    acc1 = _edge_pass(hs1, srcp, dstp)
    hs2 = _tc2_call(counts, acc1, hs1, b1, w2p)
    acc2 = _edge_pass(hs2, srcp, dstp)
    embed, hs3 = _tc3_call(counts, acc2, hs2, b2p, w3p)
    acc3 = _edge_pass(hs3, srcp, dstp)
    h_full, logp = _tc4_call(counts, acc3, hs3, b3p, wpp, bpp, wlp, blp)
    return (logp[:, :NUM_CLASSES], (h_full[:N, :3], embed[:N, :3]))


# speedup vs baseline: 93.8474x; 1.4500x over previous
import functools

import jax
import jax.numpy as jnp
from jax import lax
from jax.experimental import pallas as pl
from jax.experimental.pallas import tpu as pltpu
from jax.experimental.pallas import tpu_sc as plsc

N = 10000
E = 320000
D = 128
NUM_CLASSES = 10
F = 16
NC, NS = 2, 16
NW = NC * NS
CHUNK = 128
CPT = 79
E_PAD = NW * CPT * CHUNK
N_PAD = 10240
RPT = N_PAD // NS
LEAD = 8
NRING = 2 * LEAD
NROWS = N_PAD * F // 128
NMASK = N * F // 128

_MESH = dict(core_axis_name="c", subcore_axis_name="s")


@functools.partial(
    pl.kernel,
    out_type=jax.ShapeDtypeStruct((NC, N_PAD, F), jnp.float32),
    mesh=plsc.VectorSubcoreMesh(**_MESH),
    scratch_types=[
        pltpu.VMEM((CPT, CHUNK), jnp.int32),
        pltpu.VMEM((CHUNK,), jnp.float32),
        pltpu.VMEM((CHUNK,), jnp.float32),
        pltpu.VMEM((RPT,), jnp.float32),
        pltpu.VMEM((RPT, F), jnp.float32),
        pltpu.VMEM_SHARED((N_PAD,), jnp.float32),
        pltpu.SemaphoreType.DMA,
    ],
    compiler_params=pltpu.CompilerParams(use_tc_tiling_on_sc=False),
)
def _deg_pass(dst_hbm, out_hbm, dst_v, ones_v, zb_v, loc_v, exp_v, cnt, sem):
    c = lax.axis_index("c")
    s = lax.axis_index("s")
    wid = c * NS + s
    pltpu.sync_copy(dst_hbm.at[wid], dst_v)
    for i in range(CHUNK // F):
        ones_v[pl.ds(i * F, F)] = jnp.ones((F,), jnp.float32)
        zb_v[pl.ds(i * F, F)] = jnp.zeros((F,), jnp.float32)
    for k in range(RPT // CHUNK):
        pltpu.sync_copy(zb_v, cnt.at[pl.ds(s * RPT + k * CHUNK, CHUNK)])
    plsc.subcore_barrier()

    def body(j, carry):
        pltpu.async_copy(ones_v, cnt.at[dst_v.at[j]], sem, add=True)
        return carry

    lax.fori_loop(0, CPT, body, 0)

    def drain(j, carry):
        pltpu.make_async_copy(out_hbm.at[0, pl.ds(0, CHUNK // F)],
                              exp_v.at[pl.ds(0, CHUNK // F)], sem).wait()
        return carry

    lax.fori_loop(0, CPT, drain, 0)
    plsc.subcore_barrier()

    pltpu.sync_copy(cnt.at[pl.ds(s * RPT, RPT)], loc_v)

    def expand(i, carry):
        c16 = loc_v[pl.ds(i * F, F)]
        for k in range(F):
            exp_v[i * F + k, :] = jnp.full((F,), c16[k], jnp.float32)
        return carry

    lax.fori_loop(0, RPT // F, expand, 0)
    pltpu.sync_copy(exp_v, out_hbm.at[c, pl.ds(s * RPT, RPT)])


@functools.partial(
    pl.kernel,
    out_type=jax.ShapeDtypeStruct((NC, N_PAD, F), jnp.float32),
    mesh=plsc.VectorSubcoreMesh(**_MESH),
    scratch_types=[
        pltpu.VMEM((CPT, CHUNK), jnp.int32),
        pltpu.VMEM((CPT, CHUNK), jnp.int32),
        pltpu.VMEM((NRING, CHUNK, F), jnp.float32),
        pltpu.VMEM((CHUNK, F), jnp.float32),
        pltpu.VMEM_SHARED((N_PAD, F), jnp.float32),
        pltpu.VMEM_SHARED((N_PAD, F), jnp.float32),
        pltpu.SemaphoreType.DMA,
        pltpu.SemaphoreType.DMA,
    ],
    compiler_params=pltpu.CompilerParams(use_tc_tiling_on_sc=False),
)
def _edge_pass(h_hbm, src_hbm, dst_hbm, out_hbm,
               src_v, dst_v, rows_v, zb_v, acc, h_stage, sem_g, sem_s):
    c = lax.axis_index("c")
    s = lax.axis_index("s")
    wid = c * NS + s
    pltpu.sync_copy(src_hbm.at[wid], src_v)
    pltpu.sync_copy(dst_hbm.at[wid], dst_v)
    pltpu.sync_copy(h_hbm.at[pl.ds(s * RPT, RPT)],
                    h_stage.at[pl.ds(s * RPT, RPT)])
    for i in range(CHUNK):
        zb_v[i, :] = jnp.zeros((F,), jnp.float32)
    for k in range(RPT // CHUNK):
        pltpu.sync_copy(zb_v, acc.at[pl.ds(s * RPT + k * CHUNK, CHUNK)])
    plsc.subcore_barrier()

    for b in range(LEAD):
        pltpu.async_copy(h_stage.at[src_v.at[b]], rows_v.at[b], sem_g)

    def body(j, carry):
        pltpu.make_async_copy(out_hbm.at[0, pl.ds(0, CHUNK)],
                              rows_v.at[0], sem_g).wait()
        pltpu.async_copy(rows_v.at[j % NRING], acc.at[dst_v.at[j]],
                         sem_s, add=True)

        @pl.when(j >= LEAD)
        def _():
            pltpu.make_async_copy(out_hbm.at[0, pl.ds(0, CHUNK)],
                                  rows_v.at[0], sem_s).wait()

        @pl.when(j + LEAD < CPT)
        def _():
            jn = jnp.minimum(j + LEAD, CPT - 1)
            pltpu.async_copy(h_stage.at[src_v.at[jn]],
                             rows_v.at[jn % NRING], sem_g)
        return carry

    lax.fori_loop(0, CPT, body, 0)
    for b in range(LEAD):
        pltpu.make_async_copy(out_hbm.at[0, pl.ds(0, CHUNK)],
                              rows_v.at[0], sem_s).wait()
    plsc.subcore_barrier()
    pltpu.sync_copy(acc.at[pl.ds(s * RPT, RPT)],
                    out_hbm.at[c, pl.ds(s * RPT, RPT)])


def _dinv_fl(cnt_ref):
    c = cnt_ref[0:NROWS, :] + cnt_ref[NROWS:2 * NROWS, :]
    return lax.rsqrt(c + 1.0)


def _asum(acc_ref):
    return acc_ref[0:NROWS, :] + acc_ref[NROWS:2 * NROWS, :]


def _tc1(cnt_ref, xv_ref, w1bd_ref, hs_ref):
    h = jnp.dot(xv_ref[:, :], w1bd_ref[:, :],
                preferred_element_type=jnp.float32)
    hs_ref[:, :] = h * _dinv_fl(cnt_ref)


def _tc2(cnt_ref, acc_ref, hs1_ref, b1f_ref, w2bd_ref, hs2_ref):
    dinv = _dinv_fl(cnt_ref)
    out1 = dinv * (_asum(acc_ref) + hs1_ref[:, :]) + b1f_ref[:][None, :]
    h2 = jnp.maximum(out1, 0.0)
    hs2_ref[:, :] = jnp.dot(h2, w2bd_ref[:, :],
                            preferred_element_type=jnp.float32) * dinv


def _tc3(cnt_ref, acc_ref, hs2_ref, b2f_ref, w3bd_ref, onesbd_ref,
         embed_ref, hs3_ref):
    dinv = _dinv_fl(cnt_ref)
    out2 = dinv * (_asum(acc_ref) + hs2_ref[:, :]) + b2f_ref[:][None, :]
    ss = jnp.dot(out2 * out2, onesbd_ref[:, :],
                 preferred_element_type=jnp.float32)
    embed = out2 / jnp.maximum(jnp.sqrt(ss), 1e-12)
    embed_ref[:, :] = embed
    hs3_ref[:, :] = jnp.dot(embed, w3bd_ref[:, :],
                            preferred_element_type=jnp.float32) * dinv


def _tc4(cnt_ref, acc_ref, hs3_ref, b3f_ref, wpbd_ref, bpf_ref, wl_ref,
         bl_ref, fold_ref, h_ref, logp_ref):
    dinv = _dinv_fl(cnt_ref)
    out3 = dinv * (_asum(acc_ref) + hs3_ref[:, :]) + b3f_ref[:][None, :]
    hfull = jnp.dot(out3, wpbd_ref[:, :],
                    preferred_element_type=jnp.float32) + bpf_ref[:][None, :]
    rid = lax.broadcasted_iota(jnp.int32, (NROWS, 128), 0)
    hmask = jnp.where(rid < NMASK, hfull, 0.0)
    h_ref[:, :] = hmask
    s128 = jnp.sum(hmask, axis=0, keepdims=True)
    pooled = jnp.dot(s128, fold_ref[:, :],
                     preferred_element_type=jnp.float32) * (1.0 / N)
    logits = jnp.dot(pooled, wl_ref[:, :],
                     preferred_element_type=jnp.float32) + bl_ref[:][None, :]
    cid = lax.broadcasted_iota(jnp.int32, (1, F), 1)
    valid = cid < NUM_CLASSES
    m = jnp.max(jnp.where(valid, logits, -1e30), axis=1, keepdims=True)
    ex = jnp.where(valid, jnp.exp(logits - m), 0.0)
    lse = jnp.log(jnp.sum(ex, axis=1, keepdims=True))
    logp_ref[:, :] = logits - m - lse


_SD = jax.ShapeDtypeStruct
_FL = _SD((NROWS, 128), jnp.float32)
_tc1_call = pl.pallas_call(_tc1, out_shape=_FL)
_tc2_call = pl.pallas_call(_tc2, out_shape=_FL)
_tc3_call = pl.pallas_call(_tc3, out_shape=[_FL, _FL])
_tc4_call = pl.pallas_call(_tc4, out_shape=[_FL, _SD((1, F), jnp.float32)])


def kernel(x, edge_index, W1, b1, W2, b2, W3, b3, Wp, bp, Wl, bl):
    f32 = jnp.float32
    ei = edge_index.astype(jnp.int32)
    pad = N + (jnp.arange(E_PAD - E, dtype=jnp.int32) % (N_PAD - N))
    srcp = jnp.concatenate([ei[0], pad]).reshape(NW, CPT, CHUNK)
    dstp = jnp.concatenate([ei[1], pad]).reshape(NW, CPT, CHUNK)
    xv = jnp.pad(x, ((0, N_PAD - N), (0, 0))).reshape(N_PAD // 8, 8 * D)
    w2p = jnp.pad(W2, ((0, 0), (0, F - W2.shape[1])))
    w3p = jnp.pad(W3, ((0, F - W3.shape[0]), (0, F - W3.shape[1])))
    wpp = jnp.pad(Wp, ((0, F - Wp.shape[0]), (0, F - Wp.shape[1])))
    wlp = jnp.pad(Wl, ((0, F - Wl.shape[0]), (0, F - Wl.shape[1])))
    blp = jnp.pad(bl, (0, F - bl.shape[0]))
    eye8 = jnp.eye(8, dtype=f32)
    w1bd = jnp.kron(eye8, W1)
    w2bd = jnp.kron(eye8, w2p)
    w3bd = jnp.kron(eye8, w3p)
    wpbd = jnp.kron(eye8, wpp)
    onesbd = jnp.kron(eye8, jnp.ones((F, F), f32))
    fold = jnp.kron(jnp.ones((8, 1), f32), jnp.eye(F, dtype=f32))
    b1f = jnp.tile(b1, 8)
    b2f = jnp.tile(jnp.pad(b2, (0, F - b2.shape[0])), 8)
    b3f = jnp.tile(jnp.pad(b3, (0, F - b3.shape[0])), 8)
    bpf = jnp.tile(jnp.pad(bp, (0, F - bp.shape[0])), 8)

    cexp = _deg_pass(dstp).reshape(2 * NROWS, 128)
    hs1 = _tc1_call(cexp, xv, w1bd)
    acc1 = _edge_pass(hs1.reshape(N_PAD, F), srcp, dstp).reshape(2 * NROWS, 128)
    hs2 = _tc2_call(cexp, acc1, hs1, b1f, w2bd)
    acc2 = _edge_pass(hs2.reshape(N_PAD, F), srcp, dstp).reshape(2 * NROWS, 128)
    embed, hs3 = _tc3_call(cexp, acc2, hs2, b2f, w3bd, onesbd)
    acc3 = _edge_pass(hs3.reshape(N_PAD, F), srcp, dstp).reshape(2 * NROWS, 128)
    hm, logp = _tc4_call(cexp, acc3, hs3, b3f, wpbd, bpf, wlp, blp, fold)
    h_full = hm.reshape(N_PAD, F)
    embed_full = embed.reshape(N_PAD, F)
    return (logp[:, :NUM_CLASSES], (h_full[:N, :3], embed_full[:N, :3]))
